# Initial kernel scaffold; baseline (speedup 1.0000x reference)
import jax
import jax.numpy as jnp
from jax.experimental import pallas as pl


def kernel(indices, table):
    raise NotImplementedError("write your pallas kernel here")


# speedup vs baseline: 1.1810x; 1.1810x over previous
import functools

import jax
import jax.numpy as jnp
from jax import lax
from jax.experimental import pallas as pl
from jax.experimental.pallas import tpu as pltpu
from jax.experimental.pallas import tpu_sc as plsc

_FIELDS = 26
_DIM = 16


@functools.lru_cache(maxsize=None)
def _make_gather(n_rows: int, dim: int):
    info = plsc.get_sparse_core_info()
    nc, ns = info.num_cores, info.num_subcores
    nw = nc * ns
    assert n_rows % nw == 0
    rows_per_w = n_rows // nw
    chunk = 3328
    assert rows_per_w % chunk == 0
    n_chunks = rows_per_w // chunk

    mesh = plsc.VectorSubcoreMesh(core_axis_name="c", subcore_axis_name="s")

    @functools.partial(
        pl.kernel,
        mesh=mesh,
        out_type=jax.ShapeDtypeStruct((n_rows, dim), jnp.float32),
        scratch_types=[
            pltpu.VMEM((chunk,), jnp.int32),
            pltpu.VMEM((chunk, dim), jnp.float32),
            pltpu.SemaphoreType.DMA,
        ],
        compiler_params=pltpu.CompilerParams(use_tc_tiling_on_sc=False),
    )
    def gather_kernel(idx_hbm, table_hbm, out_hbm, idx_v, rows_v, sem):
        wid = lax.axis_index("s") * nc + lax.axis_index("c")
        base = wid * rows_per_w
        for c in range(n_chunks):
            off = base + c * chunk
            pltpu.sync_copy(idx_hbm.at[pl.ds(off, chunk)], idx_v)
            pltpu.async_copy(table_hbm.at[idx_v], rows_v, sem).wait()
            pltpu.sync_copy(rows_v, out_hbm.at[pl.ds(off, chunk)])

    return gather_kernel


def kernel(indices, table):
    batch = indices.shape[0]
    idx_flat = indices.reshape(-1).astype(jnp.int32)
    out = _make_gather(idx_flat.shape[0], _DIM)(idx_flat, table)
    return out.reshape(batch, _FIELDS * _DIM)


# speedup vs baseline: 1.1904x; 1.0080x over previous
import functools

import jax
import jax.numpy as jnp
from jax import lax
from jax.experimental import pallas as pl
from jax.experimental.pallas import tpu as pltpu
from jax.experimental.pallas import tpu_sc as plsc

_FIELDS = 26
_DIM = 16


@functools.lru_cache(maxsize=None)
def _make_gather(n_rows: int, dim: int):
    info = plsc.get_sparse_core_info()
    nc, ns = info.num_cores, info.num_subcores
    nw = nc * ns
    assert n_rows % nw == 0
    rows_per_w = n_rows // nw
    chunk = 1664
    nbuf = 4
    assert rows_per_w % chunk == 0
    n_chunks = rows_per_w // chunk

    mesh = plsc.VectorSubcoreMesh(core_axis_name="c", subcore_axis_name="s")

    @functools.partial(
        pl.kernel,
        mesh=mesh,
        out_type=jax.ShapeDtypeStruct((n_rows, dim), jnp.float32),
        scratch_types=[
            pltpu.VMEM((rows_per_w,), jnp.int32),
            [pltpu.VMEM((chunk, dim), jnp.float32) for _ in range(nbuf)],
            [pltpu.SemaphoreType.DMA for _ in range(nbuf)],
            [pltpu.SemaphoreType.DMA for _ in range(nbuf)],
        ],
        compiler_params=pltpu.CompilerParams(use_tc_tiling_on_sc=False),
    )
    def gather_kernel(idx_hbm, table_hbm, out_hbm, idx_v, rows, gsems, wsems):
        wid = lax.axis_index("s") * nc + lax.axis_index("c")
        base = wid * rows_per_w
        pltpu.sync_copy(idx_hbm.at[pl.ds(base, rows_per_w)], idx_v)

        def start_gather(c):
            b = c % nbuf
            return pltpu.async_copy(
                table_hbm.at[idx_v.at[pl.ds(c * chunk, chunk)]], rows[b], gsems[b]
            )

        gathers = [None] * n_chunks
        writes = [None] * n_chunks
        for c in range(min(nbuf - 1, n_chunks)):
            gathers[c] = start_gather(c)
        for c in range(n_chunks):
            b = c % nbuf
            nxt = c + nbuf - 1
            if nxt < n_chunks:
                prev = nxt - nbuf
                if prev >= 0:
                    writes[prev].wait()
                gathers[nxt] = start_gather(nxt)
            gathers[c].wait()
            writes[c] = pltpu.async_copy(
                rows[b], out_hbm.at[pl.ds(base + c * chunk, chunk)], wsems[b]
            )
        for c in range(max(0, n_chunks - nbuf), n_chunks):
            if writes[c] is not None:
                writes[c].wait()

    return gather_kernel


def kernel(indices, table):
    batch = indices.shape[0]
    idx_flat = indices.reshape(-1).astype(jnp.int32)
    out = _make_gather(idx_flat.shape[0], _DIM)(idx_flat, table)
    return out.reshape(batch, _FIELDS * _DIM)


# speedup vs baseline: 3.7298x; 3.1331x over previous
import functools

import jax
import jax.numpy as jnp
from jax import lax
from jax.experimental import pallas as pl
from jax.experimental.pallas import tpu as pltpu
from jax.experimental.pallas import tpu_sc as plsc

_FIELDS = 26
_DIM = 16
_W = 32768
_BR = _W * _DIM // 128
_BR_SHIFT = _BR.bit_length() - 1


@functools.lru_cache(maxsize=None)
def _make_detile_tc(vocab: int, dim: int):
    vocab_pad = -(-vocab // _W) * _W
    out_rows = vocab_pad * dim // 128

    nk = 128 // dim

    def body(in_ref, out_ref, x2):
        for k in range(nk):
            x2[k * dim:(k + 1) * dim, :] = in_ref[:, k * _BR:(k + 1) * _BR]
        out_ref[...] = jnp.transpose(x2[...])

    return pl.pallas_call(
        body,
        grid=(pl.cdiv(out_rows, _BR),),
        in_specs=[pl.BlockSpec((dim, _W), lambda i: (0, i))],
        out_specs=pl.BlockSpec((_BR, 128), lambda i: (i, 0)),
        out_shape=jax.ShapeDtypeStruct((out_rows, 128), jnp.float32),
        scratch_shapes=[pltpu.VMEM((128, _BR), jnp.float32)],
        compiler_params=pltpu.CompilerParams(
            dimension_semantics=("parallel",),
        ),
    )


@functools.lru_cache(maxsize=None)
def _make_gather(n_rows: int, vocab: int, dim: int):
    info = plsc.get_sparse_core_info()
    nc, ns = info.num_cores, info.num_subcores
    nw = nc * ns
    assert n_rows % nw == 0
    rows_per_w = n_rows // nw
    chunk = 1664
    nbuf = 4
    assert rows_per_w % chunk == 0
    n_chunks = rows_per_w // chunk

    mesh = plsc.VectorSubcoreMesh(core_axis_name="c", subcore_axis_name="s")

    @functools.partial(
        pl.kernel,
        mesh=mesh,
        out_type=jax.ShapeDtypeStruct((n_rows, dim), jnp.float32),
        scratch_types=[
            pltpu.VMEM((rows_per_w,), jnp.int32),
            [pltpu.VMEM((chunk, dim), jnp.float32) for _ in range(nbuf)],
            [pltpu.SemaphoreType.DMA for _ in range(nbuf)],
            [pltpu.SemaphoreType.DMA for _ in range(nbuf)],
        ],
        compiler_params=pltpu.CompilerParams(use_tc_tiling_on_sc=False),
    )
    def gather_kernel(idx_hbm, table_hbm, out_hbm, idx_v, rows, gsems, wsems):
        wid = lax.axis_index("s") * nc + lax.axis_index("c")
        base = wid * rows_per_w
        pltpu.sync_copy(idx_hbm.at[pl.ds(base, rows_per_w)], idx_v)

        def xform(j, carry):
            v = idx_v[pl.ds(j * 16, 16)]
            idx_v[pl.ds(j * 16, 16)] = (
                (v & -_W) + ((v & (_BR - 1)) << 3) + ((v >> _BR_SHIFT) & 7)
            )
            return carry

        lax.fori_loop(0, rows_per_w // 16, xform, 0)

        def start_gather(c):
            b = c % nbuf
            return pltpu.async_copy(
                table_hbm.at[idx_v.at[pl.ds(c * chunk, chunk)]], rows[b], gsems[b]
            )

        gathers = [None] * n_chunks
        writes = [None] * n_chunks
        for c in range(min(nbuf - 1, n_chunks)):
            gathers[c] = start_gather(c)
        for c in range(n_chunks):
            b = c % nbuf
            nxt = c + nbuf - 1
            if nxt < n_chunks:
                prev = nxt - nbuf
                if prev >= 0:
                    writes[prev].wait()
                gathers[nxt] = start_gather(nxt)
            gathers[c].wait()
            writes[c] = pltpu.async_copy(
                rows[b], out_hbm.at[pl.ds(base + c * chunk, chunk)], wsems[b]
            )
        for c in range(max(0, n_chunks - nbuf), n_chunks):
            if writes[c] is not None:
                writes[c].wait()

    return gather_kernel


def kernel(indices, table):
    batch = indices.shape[0]
    vocab, dim = table.shape
    idx_flat = indices.reshape(-1).astype(jnp.int32)
    vocab_pad = -(-vocab // _W) * _W
    tab_t = table.T
    table128 = _make_detile_tc(vocab, dim)(tab_t)
    table_lin = table128.reshape(vocab_pad, dim)
    out = _make_gather(idx_flat.shape[0], vocab_pad, dim)(idx_flat, table_lin)
    return out.reshape(batch, _FIELDS * _DIM)
